# baseline (speedup 1.0000x reference)
import jax
import jax.numpy as jnp
from jax import lax
from jax.experimental import pallas as pl
from jax.experimental.pallas import tpu as pltpu

_NUM_MOD = 3
_BLOCK = 2048


def _moe_block_kernel(x_ref, m_ref, w_ref, o_ref):
    xb = x_ref[...]
    m = m_ref[0, 0, :]
    w = w_ref[...]
    mcol = m[:, None]
    acc = None
    for i in range(_NUM_MOD):
        xi = jnp.where(mcol == i, xb, 0.0)
        yi = lax.dot_general(
            xi, w[i], (((1,), (1,)), ((), ())),
            preferred_element_type=jnp.float32)
        acc = yi if acc is None else acc + yi
    o_ref[...] = acc


def kernel(x, modality_mapping, W):
    n, h = x.shape
    b = _BLOCK
    nblk = n // b
    m3 = modality_mapping.reshape(nblk, 1, b)
    return pl.pallas_call(
        _moe_block_kernel,
        grid=(nblk,),
        in_specs=[
            pl.BlockSpec((b, h), lambda i: (i, 0)),
            pl.BlockSpec((1, 1, b), lambda i: (i, 0, 0)),
            pl.BlockSpec((_NUM_MOD, h, h), lambda i: (0, 0, 0)),
        ],
        out_specs=pl.BlockSpec((b, h), lambda i: (i, 0)),
        out_shape=jax.ShapeDtypeStruct((n, h), x.dtype),
        compiler_params=pltpu.CompilerParams(dimension_semantics=("parallel",)),
    )(x, m3, W)


# speedup vs baseline: 1.1562x; 1.1562x over previous
import jax
import jax.numpy as jnp
from jax import lax
from jax.experimental import pallas as pl

_NUM_MOD = 3
_BLOCK = 8192


def _moe_block_kernel(x_ref, m_ref, w_ref, o_ref):
    xb = x_ref[...]
    m = m_ref[0, 0, :]
    w = w_ref[...]
    ys = [
        lax.dot_general(
            xb, w[i], (((1,), (1,)), ((), ())),
            preferred_element_type=jnp.float32)
        for i in range(_NUM_MOD)
    ]
    mcol = m[:, None]
    o_ref[...] = jnp.where(mcol == 0, ys[0],
                           jnp.where(mcol == 1, ys[1], ys[2]))


def kernel(x, modality_mapping, W):
    n, h = x.shape
    b = _BLOCK
    nblk = n // b
    m3 = modality_mapping.reshape(nblk, 1, b)
    return pl.pallas_call(
        _moe_block_kernel,
        grid=(nblk,),
        in_specs=[
            pl.BlockSpec((b, h), lambda i: (i, 0)),
            pl.BlockSpec((1, 1, b), lambda i: (i, 0, 0)),
            pl.BlockSpec((_NUM_MOD, h, h), lambda i: (0, 0, 0)),
        ],
        out_specs=pl.BlockSpec((b, h), lambda i: (i, 0)),
        out_shape=jax.ShapeDtypeStruct((n, h), x.dtype),
    )(x, m3, W)
